# Initial kernel scaffold; baseline (speedup 1.0000x reference)
import jax
import jax.numpy as jnp
from jax.experimental import pallas as pl


def kernel(node_fea, table):
    raise NotImplementedError("write your pallas kernel here")


# speedup vs baseline: 8.8469x; 8.8469x over previous
import functools

import jax
import jax.numpy as jnp
from jax import lax
from jax.experimental import pallas as pl
from jax.experimental.pallas import tpu as pltpu
from jax.experimental.pallas import tpu_sc as plsc

NC, NS, L = 2, 16, 16
NW = NC * NS

B = 16384 * 26
D = 128
CHUNK = 128
B_PER_W = B // NW
N_CHUNKS = B_PER_W // CHUNK
NBUF = 2


def _emb_body(table_hbm, idx_hbm, out_hbm, idx_v, bufs, gsems, osems):
    wid = lax.axis_index("s") * NC + lax.axis_index("c")
    chunk_base = wid * N_CHUNKS
    row_base = wid * B_PER_W

    pltpu.sync_copy(idx_hbm.at[pl.ds(chunk_base, N_CHUNKS)], idx_v)

    def gather(j, b):
        return pltpu.make_async_copy(
            table_hbm.at[idx_v.at[j]], bufs[b], gsems[b])

    def writeback(j, b):
        return pltpu.make_async_copy(
            bufs[b], out_hbm.at[pl.ds(row_base + j * CHUNK, CHUNK)], osems[b])

    for b in range(NBUF):
        gather(b, b).start()

    def step(jj, _):
        for b in range(NBUF):
            j = jj * NBUF + b
            gather(j, b).wait()
            writeback(j, b).start()
            nxt = j + NBUF

            @pl.when(nxt < N_CHUNKS)
            def _():
                writeback(j, b).wait()
                gather(nxt, b).start()
        return 0

    lax.fori_loop(0, N_CHUNKS // NBUF, step, 0)

    for b in range(NBUF):
        j = N_CHUNKS - NBUF + b
        writeback(j, b).wait()


@jax.jit
def _emb_lookup(idx2d, table):
    mesh = plsc.VectorSubcoreMesh(core_axis_name="c", subcore_axis_name="s")
    f = pl.kernel(
        _emb_body,
        out_type=jax.ShapeDtypeStruct((B, D), jnp.float32),
        mesh=mesh,
        scratch_types=[
            pltpu.VMEM((N_CHUNKS, CHUNK), jnp.int32),
            [pltpu.VMEM((CHUNK, D), jnp.float32) for _ in range(NBUF)],
            [pltpu.SemaphoreType.DMA for _ in range(NBUF)],
            [pltpu.SemaphoreType.DMA for _ in range(NBUF)],
        ],
    )
    return f(table, idx2d)


def kernel(node_fea, table):
    idx2d = node_fea.astype(jnp.int32).reshape(NW * N_CHUNKS, CHUNK)
    return _emb_lookup(idx2d, table)


# speedup vs baseline: 8.9948x; 1.0167x over previous
import functools

import jax
import jax.numpy as jnp
from jax import lax
from jax.experimental import pallas as pl
from jax.experimental.pallas import tpu as pltpu
from jax.experimental.pallas import tpu_sc as plsc

NC, NS, L = 2, 16, 16
NW = NC * NS

B = 16384 * 26
D = 128
CHUNK = 128
B_PER_W = B // NW
N_CHUNKS = B_PER_W // CHUNK
NBUF = 4


def _emb_body(table_hbm, idx_hbm, out_hbm, idx_v, bufs, gsems, osems):
    wid = lax.axis_index("s") * NC + lax.axis_index("c")
    chunk_base = wid * N_CHUNKS
    row_base = wid * B_PER_W

    pltpu.sync_copy(idx_hbm.at[pl.ds(chunk_base, N_CHUNKS)], idx_v)

    def gather(j, b):
        return pltpu.make_async_copy(
            table_hbm.at[idx_v.at[j]], bufs[b], gsems[b])

    def writeback(j, b):
        return pltpu.make_async_copy(
            bufs[b], out_hbm.at[pl.ds(row_base + j * CHUNK, CHUNK)], osems[b])

    for b in range(NBUF):
        gather(b, b).start()

    def step(jj, _):
        for b in range(NBUF):
            j = jj * NBUF + b
            gather(j, b).wait()
            writeback(j, b).start()
            nxt = j + NBUF

            @pl.when(nxt < N_CHUNKS)
            def _():
                writeback(j, b).wait()
                gather(nxt, b).start()
        return 0

    lax.fori_loop(0, N_CHUNKS // NBUF, step, 0)

    for b in range(NBUF):
        j = N_CHUNKS - NBUF + b
        writeback(j, b).wait()


@jax.jit
def _emb_lookup(idx2d, table):
    mesh = plsc.VectorSubcoreMesh(core_axis_name="c", subcore_axis_name="s")
    f = pl.kernel(
        _emb_body,
        out_type=jax.ShapeDtypeStruct((B, D), jnp.float32),
        mesh=mesh,
        scratch_types=[
            pltpu.VMEM((N_CHUNKS, CHUNK), jnp.int32),
            [pltpu.VMEM((CHUNK, D), jnp.float32) for _ in range(NBUF)],
            [pltpu.SemaphoreType.DMA for _ in range(NBUF)],
            [pltpu.SemaphoreType.DMA for _ in range(NBUF)],
        ],
    )
    return f(table, idx2d)


def kernel(node_fea, table):
    idx2d = node_fea.astype(jnp.int32).reshape(NW * N_CHUNKS, CHUNK)
    return _emb_lookup(idx2d, table)
